# baseline (speedup 1.0000x reference)
import functools

import jax
import jax.numpy as jnp
from jax.experimental import pallas as pl
from jax.experimental.pallas import tpu as pltpu

_NSEG = 64
_D = 45
_V1 = 100000
_V2 = 10000
_TILE_A = 9088
_NA = 11
_ALN_A = _NA * _TILE_A
_TILE_B = 9984
_W = 128


def _mlp3_t(x, w0t, b0, w1t, b1, w2t, b2):
    h = jnp.maximum(jnp.dot(w0t, x, preferred_element_type=jnp.float32) + b0, 0.0)
    h = jnp.maximum(jnp.dot(w1t, h, preferred_element_type=jnp.float32) + b1, 0.0)
    return jnp.maximum(jnp.dot(w2t, h, preferred_element_type=jnp.float32) + b2, 0.0)


def _mlp_t(x, w0t, b0, w1t, b1, w2t, b2, w3t, b3):
    h = _mlp3_t(x, w0t, b0, w1t, b1, w2t, b2)
    return jnp.dot(w3t, h, preferred_element_type=jnp.float32) + b3


def _kernel(cx_ref, pt_ref, lg_ref, cxb_ref, pid_ref, lid_ref,
            tca_ref, tp_ref, tid_a_ref, tcb_ref, tl_ref, tid_b_ref,
            w0t_ref, b0_ref, w1t_ref, b1_ref, w2t_ref, b2_ref,
            w3t_ref, b3_ref, out_ref,
            cbuf, idbuf, acc, sem_c, sem_x, sem_i):
    i = pl.program_id(0)
    nsteps = _NA + 2

    def start(step, slot):
        @pl.when(step < _NA)
        def _():
            pltpu.make_async_copy(
                cx_ref.at[:, pl.ds(step * _TILE_A, _TILE_A)],
                cbuf.at[slot, :, pl.ds(0, _TILE_A)], sem_c.at[slot]).start()
            pltpu.make_async_copy(
                pt_ref.at[:, pl.ds(step * _TILE_A, _TILE_A)],
                cbuf.at[slot, :, pl.ds(_TILE_A, _TILE_A)], sem_x.at[slot]).start()
            pltpu.make_async_copy(
                pid_ref.at[:, pl.ds(step * _TILE_A, _TILE_A)],
                idbuf.at[slot, :, pl.ds(0, _TILE_A)], sem_i.at[slot]).start()

        @pl.when(step == _NA)
        def _():
            pltpu.make_async_copy(
                cxb_ref.at[:, pl.ds(0, _TILE_B)],
                cbuf.at[slot, :, pl.ds(0, _TILE_B)], sem_c.at[slot]).start()
            pltpu.make_async_copy(
                lg_ref.at[:, pl.ds(0, _TILE_B)],
                cbuf.at[slot, :, pl.ds(_TILE_B, _TILE_B)], sem_x.at[slot]).start()
            pltpu.make_async_copy(
                lid_ref.at[:, pl.ds(0, _TILE_B)],
                idbuf.at[slot, :, pl.ds(0, _TILE_B)], sem_i.at[slot]).start()

    def wait(step, slot):
        @pl.when(step < _NA)
        def _():
            pltpu.make_async_copy(
                cx_ref.at[:, pl.ds(0, _TILE_A)],
                cbuf.at[slot, :, pl.ds(0, _TILE_A)], sem_c.at[slot]).wait()
            pltpu.make_async_copy(
                pt_ref.at[:, pl.ds(0, _TILE_A)],
                cbuf.at[slot, :, pl.ds(_TILE_A, _TILE_A)], sem_x.at[slot]).wait()
            pltpu.make_async_copy(
                pid_ref.at[:, pl.ds(0, _TILE_A)],
                idbuf.at[slot, :, pl.ds(0, _TILE_A)], sem_i.at[slot]).wait()

        @pl.when(step == _NA)
        def _():
            pltpu.make_async_copy(
                cxb_ref.at[:, pl.ds(0, _TILE_B)],
                cbuf.at[slot, :, pl.ds(0, _TILE_B)], sem_c.at[slot]).wait()
            pltpu.make_async_copy(
                lg_ref.at[:, pl.ds(0, _TILE_B)],
                cbuf.at[slot, :, pl.ds(_TILE_B, _TILE_B)], sem_x.at[slot]).wait()
            pltpu.make_async_copy(
                lid_ref.at[:, pl.ds(0, _TILE_B)],
                idbuf.at[slot, :, pl.ds(0, _TILE_B)], sem_i.at[slot]).wait()

    slot = jax.lax.rem(i, 2)

    @pl.when(i == 0)
    def _():
        acc[...] = jnp.zeros_like(acc)
        start(0, 0)

    @pl.when(i + 1 < nsteps)
    def _():
        start(i + 1, jax.lax.rem(i + 1, 2))

    wait(i, slot)

    args = (w0t_ref[...], b0_ref[...], w1t_ref[...], b1_ref[...],
            w2t_ref[...], b2_ref[...], w3t_ref[...], b3_ref[...])

    def accum_d(d, ids, width, valid_from=None):
        seg = jax.lax.broadcasted_iota(jnp.int32, (_NSEG, width), 0)
        mask = ids == seg
        if valid_from is not None:
            lane = jax.lax.broadcasted_iota(jnp.int32, (_NSEG, width), 1)
            mask = jnp.logical_and(mask, lane >= valid_from)
        contrib = jnp.where(mask, jnp.broadcast_to(d, (_NSEG, width)), 0.0)
        acc[:, pl.ds(0, width)] += contrib

    def accum_pair(h3, w3c, ids, width):
        d3 = jax.lax.slice(h3, (0, 0), (16, width)) - \
            jax.lax.slice(h3, (0, width), (16, 2 * width))
        d = jnp.sum(d3 * w3c, axis=0, keepdims=True)
        accum_d(d, ids, width)

    def accum(c, x, ids, width, valid_from=None):
        d = _mlp_t(c, *args) - _mlp_t(x, *args)
        accum_d(d, ids, width, valid_from)

    args3 = args[:6]
    w3c = args[6].reshape(-1, 1)

    @pl.when(i < _NA)
    def _():
        h3 = _mlp3_t(cbuf[slot, :, pl.ds(0, 2 * _TILE_A)], *args3)
        accum_pair(h3, w3c, idbuf[slot, :, pl.ds(0, _TILE_A)], _TILE_A)

    @pl.when(i == _NA)
    def _():
        h3 = _mlp3_t(cbuf[slot, :, pl.ds(0, 2 * _TILE_B)], *args3)
        accum_pair(h3, w3c, idbuf[slot, :, pl.ds(0, _TILE_B)], _TILE_B)

    @pl.when(i == nsteps - 1)
    def _():
        accum(tca_ref[...], tp_ref[...], tid_a_ref[...], _W,
              valid_from=_W - (_V1 % _W))
        accum(tcb_ref[...], tl_ref[...], tid_b_ref[...], _W,
              valid_from=_W - (_V2 % _W))
        out_ref[...] = jnp.sum(acc[...], axis=1, keepdims=True)


def kernel(protein_conv_out, ligand_conv_out, complex_conv_out,
           protein_segment_ids, ligand_segment_ids,
           W0, b0, W1, b1, W2, b2, W3, b3):
    v1 = protein_conv_out.shape[0]
    v2 = ligand_conv_out.shape[0]
    cxT = complex_conv_out.T
    ptT = protein_conv_out.T
    lgT = ligand_conv_out.T
    cxbT = cxT[:, v1:v1 + _TILE_B]
    pid2 = protein_segment_ids.reshape(1, v1)
    lid2 = ligand_segment_ids.reshape(1, v2)
    tca = cxT[:, v1 - _W:v1]
    tp = ptT[:, v1 - _W:]
    tid_a = pid2[:, v1 - _W:]
    tcb = cxT[:, v1 + v2 - _W:]
    tl = lgT[:, v2 - _W:]
    tid_b = lid2[:, v2 - _W:]
    hbm = pl.BlockSpec(memory_space=pl.ANY)
    vmem = lambda a: pl.BlockSpec(a.shape, lambda i: (0,) * a.ndim)
    ws = (W0.T, b0.reshape(-1, 1), W1.T, b1.reshape(-1, 1),
          W2.T, b2.reshape(-1, 1), W3.T, b3.reshape(-1, 1))
    tails = (tca, tp, tid_a, tcb, tl, tid_b)
    out = pl.pallas_call(
        _kernel,
        grid=(_NA + 2,),
        in_specs=[hbm] * 6 + [vmem(t) for t in tails] + [vmem(w) for w in ws],
        out_specs=pl.BlockSpec((_NSEG, 1), lambda i: (0, 0)),
        out_shape=jax.ShapeDtypeStruct((_NSEG, 1), jnp.float32),
        scratch_shapes=[
            pltpu.VMEM((2, _D, 2 * _TILE_B), jnp.float32),
            pltpu.VMEM((2, 1, _TILE_B), jnp.int32),
            pltpu.VMEM((_NSEG, _TILE_B), jnp.float32),
            pltpu.SemaphoreType.DMA((2,)),
            pltpu.SemaphoreType.DMA((2,)),
            pltpu.SemaphoreType.DMA((2,)),
        ],
        compiler_params=pltpu.CompilerParams(
            dimension_semantics=("arbitrary",)),
    )(cxT, ptT, lgT, cxbT, pid2, lid2, *tails, *ws)
    return out


# speedup vs baseline: 1.0151x; 1.0151x over previous
import functools

import jax
import jax.numpy as jnp
from jax.experimental import pallas as pl
from jax.experimental.pallas import tpu as pltpu

_NSEG = 64
_D = 45
_V1 = 100000
_V2 = 10000
_TILE_A = 9088
_NA = 11
_ALN_A = _NA * _TILE_A
_TILE_B = 9984
_W = 128


def _mlp3_t(x, w0t, b0, w1t, b1, w2t, b2):
    h = jnp.maximum(jnp.dot(w0t, x, preferred_element_type=jnp.float32) + b0, 0.0)
    h = jnp.maximum(jnp.dot(w1t, h, preferred_element_type=jnp.float32) + b1, 0.0)
    return jnp.maximum(jnp.dot(w2t, h, preferred_element_type=jnp.float32) + b2, 0.0)


def _mlp_t(x, w0t, b0, w1t, b1, w2t, b2, w3t, b3):
    h = _mlp3_t(x, w0t, b0, w1t, b1, w2t, b2)
    return jnp.dot(w3t, h, preferred_element_type=jnp.float32) + b3


def _kernel(cx_ref, pt_ref, lg_ref, cxb_ref, pid_ref, lid_ref,
            tca_ref, tp_ref, tid_a_ref, tcb_ref, tl_ref, tid_b_ref,
            w0t_ref, b0_ref, w1t_ref, b1_ref, w2t_ref, b2_ref,
            w3t_ref, b3_ref, out_ref,
            cbuf, idbuf, acc, sem_c, sem_x, sem_i):
    i = pl.program_id(0)
    nsteps = _NA + 2

    def start(step, slot):
        @pl.when(step < _NA)
        def _():
            pltpu.make_async_copy(
                cx_ref.at[:, pl.ds(step * _TILE_A, _TILE_A)],
                cbuf.at[slot, :, pl.ds(0, _TILE_A)], sem_c.at[slot]).start()
            pltpu.make_async_copy(
                pt_ref.at[:, pl.ds(step * _TILE_A, _TILE_A)],
                cbuf.at[slot, :, pl.ds(_TILE_A, _TILE_A)], sem_x.at[slot]).start()
            pltpu.make_async_copy(
                pid_ref.at[:, pl.ds(step * _TILE_A, _TILE_A)],
                idbuf.at[slot, :, pl.ds(0, _TILE_A)], sem_i.at[slot]).start()

        @pl.when(step == _NA)
        def _():
            pltpu.make_async_copy(
                cxb_ref.at[:, pl.ds(0, _TILE_B)],
                cbuf.at[slot, :, pl.ds(0, _TILE_B)], sem_c.at[slot]).start()
            pltpu.make_async_copy(
                lg_ref.at[:, pl.ds(0, _TILE_B)],
                cbuf.at[slot, :, pl.ds(_TILE_B, _TILE_B)], sem_x.at[slot]).start()
            pltpu.make_async_copy(
                lid_ref.at[:, pl.ds(0, _TILE_B)],
                idbuf.at[slot, :, pl.ds(0, _TILE_B)], sem_i.at[slot]).start()

    def wait(step, slot):
        @pl.when(step < _NA)
        def _():
            pltpu.make_async_copy(
                cx_ref.at[:, pl.ds(0, _TILE_A)],
                cbuf.at[slot, :, pl.ds(0, _TILE_A)], sem_c.at[slot]).wait()
            pltpu.make_async_copy(
                pt_ref.at[:, pl.ds(0, _TILE_A)],
                cbuf.at[slot, :, pl.ds(_TILE_A, _TILE_A)], sem_x.at[slot]).wait()
            pltpu.make_async_copy(
                pid_ref.at[:, pl.ds(0, _TILE_A)],
                idbuf.at[slot, :, pl.ds(0, _TILE_A)], sem_i.at[slot]).wait()

        @pl.when(step == _NA)
        def _():
            pltpu.make_async_copy(
                cxb_ref.at[:, pl.ds(0, _TILE_B)],
                cbuf.at[slot, :, pl.ds(0, _TILE_B)], sem_c.at[slot]).wait()
            pltpu.make_async_copy(
                lg_ref.at[:, pl.ds(0, _TILE_B)],
                cbuf.at[slot, :, pl.ds(_TILE_B, _TILE_B)], sem_x.at[slot]).wait()
            pltpu.make_async_copy(
                lid_ref.at[:, pl.ds(0, _TILE_B)],
                idbuf.at[slot, :, pl.ds(0, _TILE_B)], sem_i.at[slot]).wait()

    slot = jax.lax.rem(i, 2)

    @pl.when(i == 0)
    def _():
        acc[...] = jnp.zeros_like(acc)
        start(0, 0)

    @pl.when(i + 1 < nsteps)
    def _():
        start(i + 1, jax.lax.rem(i + 1, 2))

    wait(i, slot)

    args = (w0t_ref[...], b0_ref[...], w1t_ref[...], b1_ref[...],
            w2t_ref[...], b2_ref[...], w3t_ref[...], b3_ref[...])

    def accum_d(d, ids, width, valid_from=None):
        seg = jax.lax.broadcasted_iota(jnp.int32, (_NSEG, width), 0)
        mask = ids == seg
        if valid_from is not None:
            lane = jax.lax.broadcasted_iota(jnp.int32, (_NSEG, width), 1)
            mask = jnp.logical_and(mask, lane >= valid_from)
        contrib = jnp.where(mask, jnp.broadcast_to(d, (_NSEG, width)), 0.0)
        acc[:, pl.ds(0, width)] += contrib

    def accum_pair(z, ids, width):
        d = jax.lax.slice(z, (0, 0), (1, width)) - \
            jax.lax.slice(z, (0, width), (1, 2 * width))
        accum_d(d, ids, width)

    def accum(c, x, ids, width, valid_from=None):
        d = _mlp_t(c, *args) - _mlp_t(x, *args)
        accum_d(d, ids, width, valid_from)

    @pl.when(i < _NA)
    def _():
        z = _mlp_t(cbuf[slot, :, pl.ds(0, 2 * _TILE_A)], *args)
        accum_pair(z, idbuf[slot, :, pl.ds(0, _TILE_A)], _TILE_A)

    @pl.when(i == _NA)
    def _():
        z = _mlp_t(cbuf[slot, :, pl.ds(0, 2 * _TILE_B)], *args)
        accum_pair(z, idbuf[slot, :, pl.ds(0, _TILE_B)], _TILE_B)

    @pl.when(i == nsteps - 1)
    def _():
        accum(tca_ref[...], tp_ref[...], tid_a_ref[...], _W,
              valid_from=_W - (_V1 % _W))
        accum(tcb_ref[...], tl_ref[...], tid_b_ref[...], _W,
              valid_from=_W - (_V2 % _W))
        out_ref[...] = jnp.sum(acc[...], axis=1, keepdims=True)


def kernel(protein_conv_out, ligand_conv_out, complex_conv_out,
           protein_segment_ids, ligand_segment_ids,
           W0, b0, W1, b1, W2, b2, W3, b3):
    v1 = protein_conv_out.shape[0]
    v2 = ligand_conv_out.shape[0]
    cxT = complex_conv_out.T
    ptT = protein_conv_out.T
    lgT = ligand_conv_out.T
    cxbT = cxT[:, v1:v1 + _TILE_B]
    pid2 = protein_segment_ids.reshape(1, v1)
    lid2 = ligand_segment_ids.reshape(1, v2)
    tca = cxT[:, v1 - _W:v1]
    tp = ptT[:, v1 - _W:]
    tid_a = pid2[:, v1 - _W:]
    tcb = cxT[:, v1 + v2 - _W:]
    tl = lgT[:, v2 - _W:]
    tid_b = lid2[:, v2 - _W:]
    hbm = pl.BlockSpec(memory_space=pl.ANY)
    vmem = lambda a: pl.BlockSpec(a.shape, lambda i: (0,) * a.ndim)
    ws = (W0.T, b0.reshape(-1, 1), W1.T, b1.reshape(-1, 1),
          W2.T, b2.reshape(-1, 1), W3.T, b3.reshape(-1, 1))
    tails = (tca, tp, tid_a, tcb, tl, tid_b)
    out = pl.pallas_call(
        _kernel,
        grid=(_NA + 2,),
        in_specs=[hbm] * 6 + [vmem(t) for t in tails] + [vmem(w) for w in ws],
        out_specs=pl.BlockSpec((_NSEG, 1), lambda i: (0, 0)),
        out_shape=jax.ShapeDtypeStruct((_NSEG, 1), jnp.float32),
        scratch_shapes=[
            pltpu.VMEM((2, _D, 2 * _TILE_B), jnp.float32),
            pltpu.VMEM((2, 1, _TILE_B), jnp.int32),
            pltpu.VMEM((_NSEG, _TILE_B), jnp.float32),
            pltpu.SemaphoreType.DMA((2,)),
            pltpu.SemaphoreType.DMA((2,)),
            pltpu.SemaphoreType.DMA((2,)),
        ],
        compiler_params=pltpu.CompilerParams(
            dimension_semantics=("arbitrary",)),
    )(cxT, ptT, lgT, cxbT, pid2, lid2, *tails, *ws)
    return out


# speedup vs baseline: 1.0611x; 1.0453x over previous
import jax
import jax.numpy as jnp
from jax.experimental import pallas as pl
from jax.experimental.pallas import tpu as pltpu

_NSEG = 64
_D = 45
_V1 = 100000
_V2 = 10000
_TA = 9088
_NA = 11
_TB = 9984
_R = _TB
_W = 128
_NS = 7


def _mlp_t(x, w0t, b0, w1t, b1, w2t, b2, w3t, b3):
    h = jnp.maximum(jnp.dot(w0t, x, preferred_element_type=jnp.float32) + b0, 0.0)
    h = jnp.maximum(jnp.dot(w1t, h, preferred_element_type=jnp.float32) + b1, 0.0)
    h = jnp.maximum(jnp.dot(w2t, h, preferred_element_type=jnp.float32) + b2, 0.0)
    return jnp.dot(w3t, h, preferred_element_type=jnp.float32) + b3


def _kernel(cx_ref, pt_ref, lg_ref, cxb_ref, pid_ref, lid_ref,
            tca_ref, tp_ref, tid_a_ref, tcb_ref, tl_ref, tid_b_ref,
            w0t_ref, b0_ref, w1t_ref, b1_ref, w2t_ref, b2_ref,
            w3t_ref, b3_ref, out_ref,
            cbuf, idbuf, acc, sem_c, sem_x, sem_i):
    i = pl.program_id(0)

    def start_a(tile, slot, t):
        pltpu.make_async_copy(
            cx_ref.at[:, pl.ds(tile * _TA, _TA)],
            cbuf.at[slot, :, pl.ds(2 * t * _R, _TA)], sem_c.at[slot, t]).start()
        pltpu.make_async_copy(
            pt_ref.at[:, pl.ds(tile * _TA, _TA)],
            cbuf.at[slot, :, pl.ds((2 * t + 1) * _R, _TA)],
            sem_x.at[slot, t]).start()
        pltpu.make_async_copy(
            pid_ref.at[:, pl.ds(tile * _TA, _TA)],
            idbuf.at[slot, :, pl.ds(t * _R, _TA)], sem_i.at[slot, t]).start()

    def wait_a(slot, t):
        pltpu.make_async_copy(
            cx_ref.at[:, pl.ds(0, _TA)],
            cbuf.at[slot, :, pl.ds(2 * t * _R, _TA)], sem_c.at[slot, t]).wait()
        pltpu.make_async_copy(
            pt_ref.at[:, pl.ds(0, _TA)],
            cbuf.at[slot, :, pl.ds((2 * t + 1) * _R, _TA)],
            sem_x.at[slot, t]).wait()
        pltpu.make_async_copy(
            pid_ref.at[:, pl.ds(0, _TA)],
            idbuf.at[slot, :, pl.ds(t * _R, _TA)], sem_i.at[slot, t]).wait()

    def start_b(slot):
        pltpu.make_async_copy(
            cxb_ref.at[:, pl.ds(0, _TB)],
            cbuf.at[slot, :, pl.ds(2 * _R, _TB)], sem_c.at[slot, 1]).start()
        pltpu.make_async_copy(
            lg_ref.at[:, pl.ds(0, _TB)],
            cbuf.at[slot, :, pl.ds(3 * _R, _TB)], sem_x.at[slot, 1]).start()
        pltpu.make_async_copy(
            lid_ref.at[:, pl.ds(0, _TB)],
            idbuf.at[slot, :, pl.ds(_R, _TB)], sem_i.at[slot, 1]).start()

    def wait_b(slot):
        pltpu.make_async_copy(
            cxb_ref.at[:, pl.ds(0, _TB)],
            cbuf.at[slot, :, pl.ds(2 * _R, _TB)], sem_c.at[slot, 1]).wait()
        pltpu.make_async_copy(
            lg_ref.at[:, pl.ds(0, _TB)],
            cbuf.at[slot, :, pl.ds(3 * _R, _TB)], sem_x.at[slot, 1]).wait()
        pltpu.make_async_copy(
            lid_ref.at[:, pl.ds(0, _TB)],
            idbuf.at[slot, :, pl.ds(_R, _TB)], sem_i.at[slot, 1]).wait()

    def start(step, slot):
        @pl.when(step < 5)
        def _():
            start_a(2 * step, slot, 0)
            start_a(2 * step + 1, slot, 1)

        @pl.when(step == 5)
        def _():
            start_a(10, slot, 0)
            start_b(slot)

    def wait(step, slot):
        @pl.when(step < 5)
        def _():
            wait_a(slot, 0)
            wait_a(slot, 1)

        @pl.when(step == 5)
        def _():
            wait_a(slot, 0)
            wait_b(slot)

    slot = jax.lax.rem(i, 2)

    @pl.when(i == 0)
    def _():
        acc[...] = jnp.zeros_like(acc)
        start(0, 0)

    @pl.when(i + 1 < _NS)
    def _():
        start(i + 1, jax.lax.rem(i + 1, 2))

    wait(i, slot)

    args = (w0t_ref[...], b0_ref[...], w1t_ref[...], b1_ref[...],
            w2t_ref[...], b2_ref[...], w3t_ref[...], b3_ref[...])

    def accum_d(d, ids, width, valid_from=None):
        seg = jax.lax.broadcasted_iota(jnp.int32, (_NSEG, width), 0)
        mask = ids == seg
        if valid_from is not None:
            lane = jax.lax.broadcasted_iota(jnp.int32, (_NSEG, width), 1)
            mask = jnp.logical_and(mask, lane >= valid_from)
        contrib = jnp.where(mask, jnp.broadcast_to(d, (_NSEG, width)), 0.0)
        acc[:, pl.ds(0, width)] += contrib

    def pair(slot, t, width):
        z = _mlp_t(cbuf[slot, :, pl.ds(2 * t * _R, 2 * _R)], *args)
        d = jax.lax.slice(z, (0, 0), (1, width)) - \
            jax.lax.slice(z, (0, _R), (1, _R + width))
        accum_d(d, idbuf[slot, :, pl.ds(t * _R, width)], width)

    def accum_tail(c, x, ids, valid_from):
        d = _mlp_t(c, *args) - _mlp_t(x, *args)
        accum_d(d, ids, _W, valid_from)

    @pl.when(i < 5)
    def _():
        pair(slot, 0, _TA)
        pair(slot, 1, _TA)

    @pl.when(i == 5)
    def _():
        pair(slot, 0, _TA)
        pair(slot, 1, _TB)

    @pl.when(i == _NS - 1)
    def _():
        accum_tail(tca_ref[...], tp_ref[...], tid_a_ref[...], _W - (_V1 % _W))
        accum_tail(tcb_ref[...], tl_ref[...], tid_b_ref[...], _W - (_V2 % _W))
        out_ref[...] = jnp.sum(acc[...], axis=1, keepdims=True)


def kernel(protein_conv_out, ligand_conv_out, complex_conv_out,
           protein_segment_ids, ligand_segment_ids,
           W0, b0, W1, b1, W2, b2, W3, b3):
    v1 = protein_conv_out.shape[0]
    v2 = ligand_conv_out.shape[0]
    cxT = complex_conv_out.T
    ptT = protein_conv_out.T
    lgT = ligand_conv_out.T
    cxbT = cxT[:, v1:v1 + _TB]
    pid2 = protein_segment_ids.reshape(1, v1)
    lid2 = ligand_segment_ids.reshape(1, v2)
    tca = cxT[:, v1 - _W:v1]
    tp = ptT[:, v1 - _W:]
    tid_a = pid2[:, v1 - _W:]
    tcb = cxT[:, v1 + v2 - _W:]
    tl = lgT[:, v2 - _W:]
    tid_b = lid2[:, v2 - _W:]
    hbm = pl.BlockSpec(memory_space=pl.ANY)
    vmem = lambda a: pl.BlockSpec(a.shape, lambda i: (0,) * a.ndim)
    ws = (W0.T, b0.reshape(-1, 1), W1.T, b1.reshape(-1, 1),
          W2.T, b2.reshape(-1, 1), W3.T, b3.reshape(-1, 1))
    tails = (tca, tp, tid_a, tcb, tl, tid_b)
    out = pl.pallas_call(
        _kernel,
        grid=(_NS,),
        in_specs=[hbm] * 6 + [vmem(t) for t in tails] + [vmem(w) for w in ws],
        out_specs=pl.BlockSpec((_NSEG, 1), lambda i: (0, 0)),
        out_shape=jax.ShapeDtypeStruct((_NSEG, 1), jnp.float32),
        scratch_shapes=[
            pltpu.VMEM((2, _D, 4 * _R), jnp.float32),
            pltpu.VMEM((2, 1, 2 * _R), jnp.int32),
            pltpu.VMEM((_NSEG, _R), jnp.float32),
            pltpu.SemaphoreType.DMA((2, 2)),
            pltpu.SemaphoreType.DMA((2, 2)),
            pltpu.SemaphoreType.DMA((2, 2)),
        ],
        compiler_params=pltpu.CompilerParams(
            dimension_semantics=("arbitrary",)),
    )(cxT, ptT, lgT, cxbT, pid2, lid2, *tails, *ws)
    return out
